# baseline (speedup 1.0000x reference)
import functools

import jax
import jax.numpy as jnp
from jax.experimental import pallas as pl
from jax.experimental.pallas import tpu as pltpu


def _pass1_kernel(x_ref, adj_ref, w_ref, mn_ref, m0_acc, cs_acc):
    i = pl.program_id(0)

    @pl.when(i == 0)
    def _init():
        m0_acc[...] = jnp.zeros_like(m0_acc)
        cs_acc[...] = jnp.zeros_like(cs_acc)

    adj = adj_ref[...]
    x = x_ref[...]
    m0_acc[...] += jax.lax.dot_general(
        adj, x, (((0,), (0,)), ((), ())), preferred_element_type=jnp.float32
    )
    ones_col = jnp.ones((adj.shape[0], 1), dtype=jnp.float32)
    cs_acc[...] += jax.lax.dot_general(
        adj, ones_col, (((0,), (0,)), ((), ())), preferred_element_type=jnp.float32
    )

    @pl.when(i == pl.num_programs(0) - 1)
    def _finish():
        rcol = 1.0 / jnp.maximum(cs_acc[...], 1e-12)
        mn_ref[...] = jax.lax.dot_general(
            m0_acc[...] * rcol, w_ref[...], (((1,), (0,)), ((), ())),
            preferred_element_type=jnp.float32,
        )


def _pass2_kernel(adj_ref, mn_ref, out_ref):
    adj = adj_ref[...]
    rs = jnp.sum(adj, axis=1, keepdims=True)
    adjn = adj / jnp.maximum(rs, 1e-12)
    out_ref[...] = jax.lax.dot_general(
        adjn, mn_ref[...], (((1,), (0,)), ((), ())),
        preferred_element_type=jnp.float32,
    )


def _pick_tile(n):
    for t in (10000, 5000, 4000, 2500, 2000, 1000, 500, 200, 100, 40, 8):
        if n % t == 0 and t % 8 == 0:
            return t
    return n


@functools.partial(jax.jit, static_argnames=())
def kernel(input, adj, W, anchor_mp):
    n, d_in = input.shape
    a = adj.shape[1]
    d_out = W.shape[1]
    tile = _pick_tile(n)
    grid = (n // tile,)

    mn = pl.pallas_call(
        _pass1_kernel,
        grid=grid,
        in_specs=[
            pl.BlockSpec((tile, d_in), lambda i: (i, 0)),
            pl.BlockSpec((tile, a), lambda i: (i, 0)),
            pl.BlockSpec((d_in, d_out), lambda i: (0, 0)),
        ],
        out_specs=pl.BlockSpec((a, d_out), lambda i: (0, 0)),
        out_shape=jax.ShapeDtypeStruct((a, d_out), jnp.float32),
        scratch_shapes=[
            pltpu.VMEM((a, d_in), jnp.float32),
            pltpu.VMEM((a, 1), jnp.float32),
        ],
    )(input, adj, W)

    mn = mn * jnp.asarray(anchor_mp, mn.dtype)

    out = pl.pallas_call(
        _pass2_kernel,
        grid=grid,
        in_specs=[
            pl.BlockSpec((tile, a), lambda i: (i, 0)),
            pl.BlockSpec((a, d_out), lambda i: (0, 0)),
        ],
        out_specs=pl.BlockSpec((tile, d_out), lambda i: (i, 0)),
        out_shape=jax.ShapeDtypeStruct((n, d_out), jnp.float32),
    )(adj, mn)
    return out


# speedup vs baseline: 1.1339x; 1.1339x over previous
import jax
import jax.numpy as jnp
from jax.experimental import pallas as pl
from jax.experimental.pallas import tpu as pltpu


def _fused_kernel(x_ref, adj_ref, w_ref, out_ref,
                  adjn_sc, m0_acc, cs_acc, rcol_sc, mn_sc):
    p = pl.program_id(0)
    i = pl.program_id(1)
    num_tiles = pl.num_programs(1)
    tile = adj_ref.shape[0]

    @pl.when(jnp.logical_and(p == 0, i == 0))
    def _init():
        m0_acc[...] = jnp.zeros_like(m0_acc)
        cs_acc[...] = jnp.zeros_like(cs_acc)

    @pl.when(p == 0)
    def _phase0():
        adj = adj_ref[...]
        x = x_ref[...]
        m0_acc[...] += jax.lax.dot_general(
            adj.astype(jnp.bfloat16), x.astype(jnp.bfloat16),
            (((0,), (0,)), ((), ())), preferred_element_type=jnp.float32)
        cs_acc[...] += jnp.sum(adj, axis=0, keepdims=True)
        rs = jnp.sum(adj, axis=1, keepdims=True)
        adjn = adj / jnp.maximum(rs, 1e-12)
        adjn_sc[pl.ds(i * tile, tile), :] = adjn.astype(jnp.bfloat16)

        @pl.when(i == num_tiles - 1)
        def _finish():
            rcol_sc[...] = 1.0 / jnp.maximum(cs_acc[...], 1e-12)
            mn = jax.lax.dot_general(
                m0_acc[...].astype(jnp.bfloat16), w_ref[...].astype(jnp.bfloat16),
                (((1,), (0,)), ((), ())), preferred_element_type=jnp.float32)
            mn_sc[...] = mn.astype(jnp.bfloat16)

    @pl.when(p == 1)
    def _phase1():
        adjn = adjn_sc[pl.ds(i * tile, tile), :]
        adjn2 = (adjn * rcol_sc[...]).astype(jnp.bfloat16)
        out_ref[...] = jax.lax.dot_general(
            adjn2, mn_sc[...], (((1,), (0,)), ((), ())),
            preferred_element_type=jnp.float32)


def _pick_tile(n):
    for t in (10000, 5000, 4000, 2500, 2000, 1000, 500, 200, 100, 40, 8):
        if n % t == 0 and t % 8 == 0:
            return t
    return n


def kernel(input, adj, W, anchor_mp):
    n, d_in = input.shape
    a = adj.shape[1]
    d_out = W.shape[1]
    tile = _pick_tile(n)
    num_tiles = n // tile

    w_scaled = W * jnp.asarray(anchor_mp, W.dtype)

    out = pl.pallas_call(
        _fused_kernel,
        grid=(2, num_tiles),
        in_specs=[
            pl.BlockSpec((tile, d_in), lambda p, i: (i * (1 - p), 0)),
            pl.BlockSpec((tile, a), lambda p, i: (i * (1 - p), 0)),
            pl.BlockSpec((d_in, d_out), lambda p, i: (0, 0)),
        ],
        out_specs=pl.BlockSpec((tile, d_out), lambda p, i: (i * p, 0)),
        out_shape=jax.ShapeDtypeStruct((n, d_out), jnp.float32),
        scratch_shapes=[
            pltpu.VMEM((n, a), jnp.bfloat16),
            pltpu.VMEM((a, d_in), jnp.float32),
            pltpu.VMEM((1, a), jnp.float32),
            pltpu.VMEM((1, a), jnp.float32),
            pltpu.VMEM((a, d_out), jnp.bfloat16),
        ],
    )(input, adj, w_scaled)
    return out
